# Initial kernel scaffold; baseline (speedup 1.0000x reference)
import jax
import jax.numpy as jnp
from jax.experimental import pallas as pl


def kernel(temporal_features, static_features, edge_index, Ws, bs, W1, b1, W2, b2, Wlin, blin):
    raise NotImplementedError("write your pallas kernel here")


# speedup vs baseline: 16.0584x; 16.0584x over previous
import functools

import jax
import jax.numpy as jnp
from jax import lax
from jax.experimental import pallas as pl
from jax.experimental.pallas import tpu as pltpu
from jax.experimental.pallas import tpu_sc as plsc

N = 50000
NP = 51200
E = 800000
EP = 802816
EROWS = EP // 128
BN = 2048
GRID = NP // BN

_mesh = plsc.VectorSubcoreMesh(core_axis_name="c", subcore_axis_name="s")


def _deg_body(edges, out, dstv, ones_v, zb, accd):
    c = lax.axis_index("c")
    s = lax.axis_index("s")

    def fill_ones(i, _):
        ones_v[i, pl.ds(0, 16)] = jnp.ones((16,), jnp.float32)
        return 0

    lax.fori_loop(0, 128, fill_ones, 0)

    def fill_zero(i, _):
        zb[i, pl.ds(0, 16)] = jnp.zeros((16,), jnp.float32)
        return 0

    lax.fori_loop(0, 320, fill_zero, 0)

    tile_rows = NP // 16

    def zero_acc(m, _):
        pltpu.sync_copy(zb, accd.at[pl.ds(s * tile_rows + m * 320, 320)])
        return 0

    lax.fori_loop(0, tile_rows // 320, zero_acc, 0)
    plsc.subcore_barrier()

    w = s * 2 + c
    base = w * 196

    def step(g, _):
        row0 = base + g * 4
        pltpu.sync_copy(edges.at[1, pl.ds(row0, 4), :], dstv)
        for j in range(4):
            pltpu.sync_copy(ones_v, accd.at[dstv.at[j]], add=True)
        return 0

    lax.fori_loop(0, 49, step, 0)
    plsc.subcore_barrier()

    r0 = s * tile_rows

    def wb(m, _):
        pltpu.sync_copy(accd.at[pl.ds(r0 + m * 320, 320)], zb)
        pltpu.sync_copy(zb, out.at[c, pl.ds(r0 + m * 320, 320), :])
        return 0

    lax.fori_loop(0, tile_rows // 320, wb, 0)


_deg_call = pl.kernel(
    _deg_body,
    mesh=_mesh,
    out_type=jax.ShapeDtypeStruct((2, NP, 16), jnp.float32),
    scratch_types=[
        pltpu.VMEM((4, 128), jnp.int32),
        pltpu.VMEM((128, 16), jnp.float32),
        pltpu.VMEM((320, 16), jnp.float32),
        pltpu.VMEM_SHARED((NP, 16), jnp.float32),
    ],
    compiler_params=pltpu.CompilerParams(use_tc_tiling_on_sc=False),
)


def _edge_body(hs, edges, out, srcv, dstv, rows, zb, wbuf, accm, sem):
    c = lax.axis_index("c")
    s = lax.axis_index("s")

    def fill_zero(i, _):
        zb[i, pl.ds(0, 16)] = jnp.zeros((16,), jnp.float32)
        return 0

    lax.fori_loop(0, 320, fill_zero, 0)

    tile_rows = NP // 16
    base = s * 392
    r0 = s * tile_rows

    for p in range(2):
        def zero_acc(m, _):
            pltpu.sync_copy(zb, accm.at[pl.ds(r0 + m * 320, 320)])
            return 0

        lax.fori_loop(0, tile_rows // 320, zero_acc, 0)
        plsc.subcore_barrier()

        off = (2 * c + p) * NP

        def step(g, _):
            row0 = base + g * 8
            pltpu.sync_copy(edges.at[0, pl.ds(row0, 8), :], srcv)
            pltpu.sync_copy(edges.at[1, pl.ds(row0, 8), :], dstv)
            for j in range(8):
                for i in range(8):
                    srcv[j, pl.ds(i * 16, 16)] = srcv[j, pl.ds(i * 16, 16)] + off
            cps = [pltpu.async_copy(hs.at[srcv.at[j]], rows.at[j], sem)
                   for j in range(8)]
            for cp in cps:
                cp.wait()
            for j in range(8):
                pltpu.sync_copy(rows.at[j], accm.at[dstv.at[j]], add=True)
            return 0

        lax.fori_loop(0, 49, step, 0)
        plsc.subcore_barrier()

        def wb(m, _):
            pltpu.sync_copy(accm.at[pl.ds(r0 + m * 320, 320)], wbuf)
            pltpu.sync_copy(wbuf, out.at[2 * c + p, pl.ds(r0 + m * 320, 320), :])
            return 0

        lax.fori_loop(0, tile_rows // 320, wb, 0)
        plsc.subcore_barrier()


_edge_call = pl.kernel(
    _edge_body,
    mesh=_mesh,
    out_type=jax.ShapeDtypeStruct((4, NP, 16), jnp.float32),
    scratch_types=[
        pltpu.VMEM((8, 128), jnp.int32),
        pltpu.VMEM((8, 128), jnp.int32),
        pltpu.VMEM((8, 128, 16), jnp.float32),
        pltpu.VMEM((320, 16), jnp.float32),
        pltpu.VMEM((320, 16), jnp.float32),
        pltpu.VMEM_SHARED((NP, 16), jnp.float32),
        pltpu.SemaphoreType.DMA,
    ],
    compiler_params=pltpu.CompilerParams(use_tc_tiling_on_sc=False),
)


def _fuse_body(t_ref, s_ref, ws_ref, bs_ref, w1a_ref, w1b_ref, h1_ref):
    ps = jnp.maximum(
        jnp.dot(s_ref[...], ws_ref[...], preferred_element_type=jnp.float32)
        + bs_ref[...], 0.0)
    h1_ref[...] = (
        jnp.dot(t_ref[...], w1a_ref[...], preferred_element_type=jnp.float32)
        + jnp.dot(ps, w1b_ref[...], preferred_element_type=jnp.float32))


def _fuse_call(tf, sf, Ws, bs, W1a, W1b):
    return pl.pallas_call(
        _fuse_body,
        grid=(GRID,),
        in_specs=[
            pl.BlockSpec((BN, 64), lambda i: (i, 0)),
            pl.BlockSpec((BN, 128), lambda i: (i, 0)),
            pl.BlockSpec((128, 32), lambda i: (0, 0)),
            pl.BlockSpec((32,), lambda i: (0,)),
            pl.BlockSpec((64, 64), lambda i: (0, 0)),
            pl.BlockSpec((32, 64), lambda i: (0, 0)),
        ],
        out_specs=pl.BlockSpec((BN, 64), lambda i: (i, 0)),
        out_shape=jax.ShapeDtypeStruct((NP, 64), jnp.float32),
    )(tf, sf, Ws, bs, W1a, W1b)


def _scale_body(degp_ref, h1_ref, dinv_ref, hs_ref):
    deg = degp_ref[0, :, 0:1] + degp_ref[1, :, 0:1] + 1.0
    dinv = lax.rsqrt(deg)
    dinv_ref[...] = dinv
    hsv = h1_ref[...] * dinv
    for q in range(4):
        hs_ref[q] = hsv[:, 16 * q:16 * (q + 1)]


def _scale_call(degp, h1):
    return pl.pallas_call(
        _scale_body,
        grid=(GRID,),
        in_specs=[
            pl.BlockSpec((2, BN, 16), lambda i: (0, i, 0)),
            pl.BlockSpec((BN, 64), lambda i: (i, 0)),
        ],
        out_specs=[
            pl.BlockSpec((BN, 1), lambda i: (i, 0)),
            pl.BlockSpec((4, BN, 16), lambda i: (0, i, 0)),
        ],
        out_shape=[
            jax.ShapeDtypeStruct((NP, 1), jnp.float32),
            jax.ShapeDtypeStruct((4, NP, 16), jnp.float32),
        ],
    )(degp, h1)


def _epi1_body(acc_ref, h1_ref, dinv_ref, b1_ref, w2_ref, h2_ref, hs2_ref):
    dinv = dinv_ref[...]
    accc = jnp.concatenate([acc_ref[q] for q in range(4)], axis=1)
    x2 = jnp.maximum(accc * dinv + h1_ref[...] * (dinv * dinv) + b1_ref[...], 0.0)
    h2 = jnp.dot(x2, w2_ref[...], preferred_element_type=jnp.float32)
    h2_ref[...] = h2
    hs2 = h2 * dinv
    for q in range(4):
        hs2_ref[q] = hs2[:, 16 * q:16 * (q + 1)]


def _epi1_call(acc1, h1, dinv, b1, W2):
    return pl.pallas_call(
        _epi1_body,
        grid=(GRID,),
        in_specs=[
            pl.BlockSpec((4, BN, 16), lambda i: (0, i, 0)),
            pl.BlockSpec((BN, 64), lambda i: (i, 0)),
            pl.BlockSpec((BN, 1), lambda i: (i, 0)),
            pl.BlockSpec((64,), lambda i: (0,)),
            pl.BlockSpec((64, 64), lambda i: (0, 0)),
        ],
        out_specs=[
            pl.BlockSpec((BN, 64), lambda i: (i, 0)),
            pl.BlockSpec((4, BN, 16), lambda i: (0, i, 0)),
        ],
        out_shape=[
            jax.ShapeDtypeStruct((NP, 64), jnp.float32),
            jax.ShapeDtypeStruct((4, NP, 16), jnp.float32),
        ],
    )(acc1, h1, dinv, b1, W2)


def _epi2_body(acc_ref, h2_ref, dinv_ref, b2_ref, wlin_ref, blin_ref, out_ref):
    dinv = dinv_ref[...]
    accc = jnp.concatenate([acc_ref[q] for q in range(4)], axis=1)
    x3 = jnp.maximum(accc * dinv + h2_ref[...] * (dinv * dinv) + b2_ref[...], 0.0)
    out_ref[...] = (
        jnp.dot(x3, wlin_ref[...], preferred_element_type=jnp.float32)
        + blin_ref[...])


def _epi2_call(acc2, h2, dinv, b2, Wlin, blin):
    return pl.pallas_call(
        _epi2_body,
        grid=(GRID,),
        in_specs=[
            pl.BlockSpec((4, BN, 16), lambda i: (0, i, 0)),
            pl.BlockSpec((BN, 64), lambda i: (i, 0)),
            pl.BlockSpec((BN, 1), lambda i: (i, 0)),
            pl.BlockSpec((64,), lambda i: (0,)),
            pl.BlockSpec((64, 1), lambda i: (0, 0)),
            pl.BlockSpec((1,), lambda i: (0,)),
        ],
        out_specs=pl.BlockSpec((BN, 1), lambda i: (i, 0)),
        out_shape=jax.ShapeDtypeStruct((NP, 1), jnp.float32),
    )(acc2, h2, dinv, b2, Wlin, blin)


def kernel(temporal_features, static_features, edge_index, Ws, bs, W1, b1, W2,
           b2, Wlin, blin):
    tf = jnp.pad(temporal_features, ((0, NP - N), (0, 0)))
    sf = jnp.pad(static_features, ((0, NP - N), (0, 0)))
    ei = edge_index.astype(jnp.int32)
    pad_e = jnp.full((2, EP - E), N, jnp.int32)
    ei_p = jnp.concatenate([ei, pad_e], axis=1).reshape(2, EROWS, 128)
    W1a, W1b = W1[:64], W1[64:]

    degp = _deg_call(ei_p)
    h1 = _fuse_call(tf, sf, Ws, bs, W1a, W1b)
    dinv, hs1 = _scale_call(degp, h1)
    acc1 = _edge_call(hs1.reshape(4 * NP, 16), ei_p)
    h2, hs2 = _epi1_call(acc1, h1, dinv, b1, W2)
    acc2 = _edge_call(hs2.reshape(4 * NP, 16), ei_p)
    outp = _epi2_call(acc2, h2, dinv, b2, Wlin, blin)
    return outp[:N, 0]
